# baseline (speedup 1.0000x reference)
import functools

import jax
import jax.numpy as jnp
from jax import lax
from jax.experimental import pallas as pl
from jax.experimental.pallas import tpu as pltpu
from jax.experimental.pallas import tpu_sc as plsc

_info = plsc.get_sparse_core_info()
_NC, _NS = _info.num_cores, _info.num_subcores
_NW = _NC * _NS


@functools.cache
def _make_lookup(N, S, D):
    ipw = N // _NW
    mesh = plsc.VectorSubcoreMesh(core_axis_name="c", subcore_axis_name="s")
    assert S % 2 == 0 and ipw % 8 == 0 and ipw <= 128

    half = ipw // 2
    nchunks = 2 * S

    @functools.partial(
        pl.kernel,
        out_type=jax.ShapeDtypeStruct((S, N, D), jnp.float32),
        mesh=mesh,
        scratch_types=[
            pltpu.VMEM((S, ipw), jnp.int32),
            pltpu.VMEM((half, D), jnp.float32),
            pltpu.VMEM((half, D), jnp.float32),
            pltpu.VMEM((half, D), jnp.float32),
            pltpu.VMEM((half, D), jnp.float32),
            pltpu.VMEM((half, D), jnp.float32),
            pltpu.SemaphoreType.DMA,
            pltpu.SemaphoreType.DMA,
            pltpu.SemaphoreType.DMA,
            pltpu.SemaphoreType.DMA,
            pltpu.SemaphoreType.DMA,
            pltpu.SemaphoreType.DMA,
            pltpu.SemaphoreType.DMA,
            pltpu.SemaphoreType.DMA,
            pltpu.SemaphoreType.DMA,
            pltpu.SemaphoreType.DMA,
        ],
    )
    def lookup(
        idxt_hbm, table_hbm, out_hbm, idx_v,
        buf0, buf1, buf2, buf3, buf4,
        g0, g1, g2, g3, g4, o0, o1, o2, o3, o4,
    ):
        wid = lax.axis_index("s") * _NC + lax.axis_index("c")
        col0 = wid * ipw
        pltpu.sync_copy(idxt_hbm.at[:, pl.ds(col0, ipw)], idx_v)
        bufs = (buf0, buf1, buf2, buf3, buf4)
        gsems = (g0, g1, g2, g3, g4)
        osems = (o0, o1, o2, o3, o4)

        def idx_slice(j, h):
            return idx_v.at[j].at[pl.ds(h * half, half)]

        def out_slice(j, h):
            return out_hbm.at[j].at[pl.ds(col0 + h * half, half)]

        def start_gather(j, h, b):
            pltpu.async_copy(table_hbm.at[idx_slice(j, h)], bufs[b], gsems[b])

        def wait_gather(j, h, b):
            pltpu.make_async_copy(
                table_hbm.at[idx_slice(j, h)], bufs[b], gsems[b]
            ).wait()

        def start_out(j, h, b):
            pltpu.async_copy(bufs[b], out_slice(j, h), osems[b])

        def wait_out(j, h, b):
            pltpu.make_async_copy(bufs[b], out_slice(j, h), osems[b]).wait()

        start_gather(0, 0, 0)
        start_gather(0, 1, 1)
        start_gather(1, 0, 2)

        def deca(q, carry):
            for u in range(10):
                j = 5 * q + (u // 2)
                h = u % 2
                b = u % 5
                wait_gather(j, h, b)
                start_out(j, h, b)
                b2 = (u + 3) % 5
                jn, hn = 5 * q + ((u + 3) // 2), (u + 3) % 2
                jw, hw = 5 * q + ((u - 2) // 2), (u - 2) % 2

                if u < 2:
                    @pl.when(q == 0)
                    def _():
                        start_gather(jn, hn, b2)

                    @pl.when(q > 0)
                    def _():
                        wait_out(jw, hw, b2)
                        start_gather(jn, hn, b2)
                else:

                    @pl.when(2 * jn + hn < nchunks)
                    def _():
                        wait_out(jw, hw, b2)
                        start_gather(jn, hn, b2)

            return carry

        lax.fori_loop(0, nchunks // 10, deca, None)
        for c in range(nchunks - 5, nchunks):
            wait_out(c // 2, c % 2, c % 5)

    return lookup


def kernel(action_indices, table):
    n, s = action_indices.shape
    D = table.shape[1]
    idx_t = jnp.transpose(action_indices.astype(jnp.int32))
    out_t = _make_lookup(n, s, D)(idx_t, table)
    return jnp.transpose(out_t, (1, 0, 2))
